# baseline (speedup 1.0000x reference)
import functools

import jax
import jax.numpy as jnp
from jax import lax
from jax.experimental import pallas as pl
from jax.experimental.pallas import tpu as pltpu
from jax.experimental.pallas import tpu_sc as plsc

NC = 2
NS = 16
NW = NC * NS
L = 16
SEG = 64
VLB_WEIGHT = 0.001

_LN2 = 0.6931471805599453
_SQRT2 = 1.4142135623730951


def _vlog(x):
    xi = lax.bitcast_convert_type(x, jnp.int32)
    e = lax.shift_right_logical(xi, 23) - 127
    mi = lax.bitwise_or(lax.bitwise_and(xi, 0x007FFFFF), 0x3F800000)
    m = lax.bitcast_convert_type(mi, jnp.float32)
    big = m > _SQRT2
    m = jnp.where(big, m * 0.5, m)
    ef = e.astype(jnp.float32) + jnp.where(big, 1.0, 0.0)
    t = (m - 1.0) / (m + 1.0)
    t2 = t * t
    p = 2.0 * t * (1.0 + t2 * (1.0 / 3.0 + t2 * (1.0 / 5.0 + t2 * (1.0 / 7.0))))
    return ef * _LN2 + p


def _contrib(e, n, m, tm, v, tv):
    d = e - n
    dm = m - tm
    klh = _vlog(tv / v) + (v + dm * dm) / tv - 1.0
    return 0.5 * (d * d) + (0.25 * VLB_WEIGHT) * klh


def _sc_partials_kernel(NE, CHE):
    NV = CHE // L

    mesh = plsc.VectorSubcoreMesh(core_axis_name="c", subcore_axis_name="s")

    @functools.partial(
        pl.kernel,
        out_type=jax.ShapeDtypeStruct((2, NW, SEG * L), jnp.float32),
        mesh=mesh,
        scratch_types=[pltpu.VMEM((CHE,), jnp.float32) for _ in range(6)] + [
            pltpu.VMEM((CHE,), jnp.int32),
            pltpu.VMEM((SEG * L,), jnp.float32),
            pltpu.VMEM((SEG * L,), jnp.float32),
        ],
    )
    def k(ep, nt, mu, tmu, va, tva, ebat, part,
          ep_v, nt_v, mu_v, tmu_v, va_v, tva_v, bat_v, acc_v, cnt_v):
        wid = lax.axis_index("s") * NC + lax.axis_index("c")
        elo = wid * CHE
        base = jnp.minimum(elo, NE - CHE)
        base = pl.multiple_of(base, L)

        for src, dst in ((ep, ep_v), (nt, nt_v), (mu, mu_v), (tmu, tmu_v),
                         (va, va_v), (tva, tva_v)):
            pltpu.sync_copy(src.at[pl.ds(base, CHE)], dst)
        pltpu.sync_copy(ebat.at[pl.ds(base, CHE)], bat_v)

        zero = jnp.zeros((L,), jnp.float32)

        def zinit(g, carry):
            goff = pl.multiple_of(g * L, L)
            acc_v[pl.ds(goff, L)] = zero
            cnt_v[pl.ds(goff, L)] = zero
            return carry

        lax.fori_loop(0, SEG, zinit, 0)

        sk = lax.div(elo - base, L)

        def step(i, carry):
            off = pl.multiple_of(i * L, L)
            cel = _contrib(ep_v[pl.ds(off, L)], nt_v[pl.ds(off, L)],
                           mu_v[pl.ds(off, L)], tmu_v[pl.ds(off, L)],
                           va_v[pl.ds(off, L)], tva_v[pl.ds(off, L)])
            idv = bat_v[pl.ds(off, L)]
            validf = jnp.where(i >= sk, 1.0, 0.0)
            cel = cel * validf

            for t in range(0, L, 2):
                idt = idv[t]
                if t == 0:
                    newf = validf
                else:
                    newf = jnp.where(idt != idv[t - 2], validf, 0.0)
                mf = jnp.where(idv == idt, newf, 0.0)
                goff = pl.multiple_of(idt * L, L)
                a = acc_v[pl.ds(goff, L)]
                acc_v[pl.ds(goff, L)] = a + cel * mf
                c = cnt_v[pl.ds(goff, L)]
                cnt_v[pl.ds(goff, L)] = c + mf
            return carry

        lax.fori_loop(0, NV, step, 0)

        pltpu.sync_copy(acc_v, part.at[0, wid])
        pltpu.sync_copy(cnt_v, part.at[1, wid])

    return k


def _combine_kernel(p_ref, o_ref):
    p = p_ref[...].reshape(2, NW, SEG, L)
    s = jnp.sum(p[0], axis=(0, 2))
    c = jnp.sum(p[1], axis=(0, 2))
    o_ref[...] = (2.0 * s) / jnp.maximum(c, 1.0)


def kernel(eps_pred, noise_target, mean, true_mean, variance, true_variance,
           batch):
    N = eps_pred.shape[0]
    NE = 2 * N
    CHE = ((NE + NW * L - 1) // (NW * L)) * L

    ebat = jnp.repeat(batch, 2)
    partials = _sc_partials_kernel(NE, CHE)(
        eps_pred.reshape(-1), noise_target.reshape(-1), mean.reshape(-1),
        true_mean.reshape(-1), variance.reshape(-1),
        true_variance.reshape(-1), ebat)

    return pl.pallas_call(
        _combine_kernel,
        out_shape=jax.ShapeDtypeStruct((SEG,), jnp.float32),
    )(partials)


# speedup vs baseline: 6.1571x; 6.1571x over previous
import functools

import jax
import jax.numpy as jnp
from jax import lax
from jax.experimental import pallas as pl
from jax.experimental.pallas import tpu as pltpu
from jax.experimental.pallas import tpu_sc as plsc

NC = 2
NS = 16
NW = NC * NS
L = 16
SEG = 64
VLB_WEIGHT = 0.001

_LN2 = 0.6931471805599453
_SQRT2 = 1.4142135623730951


def _vlog(x):
    xi = lax.bitcast_convert_type(x, jnp.int32)
    e = lax.shift_right_logical(xi, 23) - 127
    mi = lax.bitwise_or(lax.bitwise_and(xi, 0x007FFFFF), 0x3F800000)
    m = lax.bitcast_convert_type(mi, jnp.float32)
    big = m > _SQRT2
    m = jnp.where(big, m * 0.5, m)
    ef = e.astype(jnp.float32) + jnp.where(big, 1.0, 0.0)
    t = (m - 1.0) / (m + 1.0)
    t2 = t * t
    p = 2.0 * t * (1.0 + t2 * (1.0 / 3.0 + t2 * (1.0 / 5.0 + t2 * (1.0 / 7.0))))
    return ef * _LN2 + p


def _contrib(e, n, m, tm, v, tv):
    d = e - n
    dm = m - tm
    klh = _vlog(tv / v) + (v + dm * dm) / tv - 1.0
    return 0.5 * (d * d) + (0.25 * VLB_WEIGHT) * klh


def _sc_partials_kernel(N, CHN):
    NV = CHN // L

    mesh = plsc.VectorSubcoreMesh(core_axis_name="c", subcore_axis_name="s")

    @functools.partial(
        pl.kernel,
        out_type=jax.ShapeDtypeStruct((2, NW, SEG * L), jnp.float32),
        mesh=mesh,
        scratch_types=[pltpu.VMEM((CHN,), jnp.float32) for _ in range(12)] + [
            pltpu.VMEM((CHN,), jnp.int32),
            pltpu.VMEM((SEG * L,), jnp.float32),
            pltpu.VMEM((SEG * L,), jnp.float32),
        ],
    )
    def k(cols, bat, part,
          ep0, nt0, mu0, tmu0, va0, tva0, ep1, nt1, mu1, tmu1, va1, tva1,
          bat_v, acc_v, cnt_v):
        wid = lax.axis_index("s") * NC + lax.axis_index("c")
        nlo = wid * CHN
        base = jnp.minimum(nlo, N - CHN)
        base = pl.multiple_of(base, L)

        streams = (ep0, ep1, nt0, nt1, mu0, mu1, tmu0, tmu1, va0, va1,
                   tva0, tva1)
        for j, dst in enumerate(streams):
            pltpu.sync_copy(cols.at[pl.ds(j * N + base, CHN)], dst)
        pltpu.sync_copy(bat.at[pl.ds(base, CHN)], bat_v)

        zero = jnp.zeros((L,), jnp.float32)

        def zinit(g, carry):
            goff = pl.multiple_of(g * L, L)
            acc_v[pl.ds(goff, L)] = zero
            cnt_v[pl.ds(goff, L)] = zero
            return carry

        lax.fori_loop(0, SEG, zinit, 0)

        sk = lax.div(nlo - base, L)

        def step(i, carry):
            off = pl.multiple_of(i * L, L)
            c0 = _contrib(ep0[pl.ds(off, L)], nt0[pl.ds(off, L)],
                          mu0[pl.ds(off, L)], tmu0[pl.ds(off, L)],
                          va0[pl.ds(off, L)], tva0[pl.ds(off, L)])
            c1 = _contrib(ep1[pl.ds(off, L)], nt1[pl.ds(off, L)],
                          mu1[pl.ds(off, L)], tmu1[pl.ds(off, L)],
                          va1[pl.ds(off, L)], tva1[pl.ds(off, L)])
            idv = bat_v[pl.ds(off, L)]
            validf = jnp.where(i >= sk, 1.0, 0.0)
            cnode = (c0 + c1) * validf

            first = idv[0]
            mf0 = jnp.where(idv == first, validf, 0.0)
            goff0 = pl.multiple_of(first * L, L)
            a0 = acc_v[pl.ds(goff0, L)]
            acc_v[pl.ds(goff0, L)] = a0 + cnode * mf0
            c0v = cnt_v[pl.ds(goff0, L)]
            cnt_v[pl.ds(goff0, L)] = c0v + mf0

            @pl.when(idv[L - 1] != first)
            def _tail():
                for t in range(1, L):
                    idt = idv[t]
                    newf = jnp.where(idt != idv[t - 1], validf, 0.0)
                    mf = jnp.where(idv == idt, newf, 0.0)
                    goff = pl.multiple_of(idt * L, L)
                    a = acc_v[pl.ds(goff, L)]
                    acc_v[pl.ds(goff, L)] = a + cnode * mf
                    c = cnt_v[pl.ds(goff, L)]
                    cnt_v[pl.ds(goff, L)] = c + mf

            return carry

        lax.fori_loop(0, NV, step, 0)

        pltpu.sync_copy(acc_v, part.at[0, wid])
        pltpu.sync_copy(cnt_v, part.at[1, wid])

    return k


def _combine_kernel(p_ref, o_ref):
    p = p_ref[...].reshape(2, NW, SEG, L)
    s = jnp.sum(p[0], axis=(0, 2))
    c = jnp.sum(p[1], axis=(0, 2))
    o_ref[...] = s / jnp.maximum(c, 1.0)


def kernel(eps_pred, noise_target, mean, true_mean, variance, true_variance,
           batch):
    N = eps_pred.shape[0]
    CHN = ((N + NW * L - 1) // (NW * L)) * L

    cols = jnp.concatenate(
        [a.T.reshape(-1) for a in (eps_pred, noise_target, mean, true_mean,
                                   variance, true_variance)])
    partials = _sc_partials_kernel(N, CHN)(cols, batch)

    return pl.pallas_call(
        _combine_kernel,
        out_shape=jax.ShapeDtypeStruct((SEG,), jnp.float32),
    )(partials)


# speedup vs baseline: 7.2092x; 1.1709x over previous
import functools

import jax
import jax.numpy as jnp
from jax import lax
from jax.experimental import pallas as pl
from jax.experimental.pallas import tpu as pltpu
from jax.experimental.pallas import tpu_sc as plsc

NC = 2
NS = 16
NW = NC * NS
L = 16
SEG = 64
NSL = SEG + 1
NSTR = 13
VLB_WEIGHT = 0.001

_LN2 = 0.6931471805599453
_SQRT2 = 1.4142135623730951


def _vlog(x):
    xi = lax.bitcast_convert_type(x, jnp.int32)
    e = lax.shift_right_logical(xi, 23) - 127
    mi = lax.bitwise_or(lax.bitwise_and(xi, 0x007FFFFF), 0x3F800000)
    m = lax.bitcast_convert_type(mi, jnp.float32)
    big = m > _SQRT2
    m = jnp.where(big, m * 0.5, m)
    ef = e.astype(jnp.float32) + jnp.where(big, 1.0, 0.0)
    t = (m - 1.0) / (m + 1.0)
    t2 = t * t
    p = 2.0 * t * (1.0 + t2 * (1.0 / 3.0 + t2 * (1.0 / 5.0 + t2 * (1.0 / 7.0))))
    return ef * _LN2 + p


def _contrib(e, n, m, tm, v, tv):
    d = e - n
    dm = m - tm
    klh = _vlog(tv / v) + (v + dm * dm) / tv - 1.0
    return 0.5 * (d * d) + (0.25 * VLB_WEIGHT) * klh


def _sc_partials_kernel(CHN):
    NV = CHN // L
    BLK = NSTR * CHN

    mesh = plsc.VectorSubcoreMesh(core_axis_name="c", subcore_axis_name="s")

    @functools.partial(
        pl.kernel,
        out_type=jax.ShapeDtypeStruct((2, NW, NSL * L), jnp.float32),
        mesh=mesh,
        scratch_types=[
            pltpu.VMEM((BLK,), jnp.float32),
            pltpu.VMEM((NSL * L,), jnp.float32),
            pltpu.VMEM((NSL * L,), jnp.float32),
        ],
    )
    def k(cols, part, blk_v, acc_v, cnt_v):
        wid = lax.axis_index("s") * NC + lax.axis_index("c")
        pltpu.sync_copy(cols.at[pl.ds(wid * BLK, BLK)], blk_v)

        zero = jnp.zeros((L,), jnp.float32)

        def zinit(g, carry):
            goff = pl.multiple_of(g * L, L)
            acc_v[pl.ds(goff, L)] = zero
            cnt_v[pl.ds(goff, L)] = zero
            return carry

        lax.fori_loop(0, NSL, zinit, 0)

        def step(i, carry):
            off = pl.multiple_of(i * L, L)

            def s(j):
                return blk_v[pl.ds(j * CHN + off, L)]

            c0 = _contrib(s(0), s(2), s(4), s(6), s(8), s(10))
            c1 = _contrib(s(1), s(3), s(5), s(7), s(9), s(11))
            idv = lax.bitcast_convert_type(s(12), jnp.int32)
            cnode = c0 + c1

            first = idv[0]
            mf0 = jnp.where(idv == first, 1.0, 0.0)
            goff0 = pl.multiple_of(first * L, L)
            a0 = acc_v[pl.ds(goff0, L)]
            acc_v[pl.ds(goff0, L)] = a0 + cnode * mf0
            c0v = cnt_v[pl.ds(goff0, L)]
            cnt_v[pl.ds(goff0, L)] = c0v + mf0

            @pl.when(idv[L - 1] != first)
            def _tail():
                for t in range(1, L):
                    idt = idv[t]
                    newf = jnp.where(idt != idv[t - 1], 1.0, 0.0)
                    mf = jnp.where(idv == idt, newf, 0.0)
                    goff = pl.multiple_of(idt * L, L)
                    a = acc_v[pl.ds(goff, L)]
                    acc_v[pl.ds(goff, L)] = a + cnode * mf
                    c = cnt_v[pl.ds(goff, L)]
                    cnt_v[pl.ds(goff, L)] = c + mf

            return carry

        lax.fori_loop(0, NV, step, 0)

        pltpu.sync_copy(acc_v, part.at[0, wid])
        pltpu.sync_copy(cnt_v, part.at[1, wid])

    return k


def _combine_kernel(p_ref, o_ref):
    p = p_ref[...].reshape(2, NW, NSL, L)
    s = jnp.sum(p[0, :, :SEG, :], axis=(0, 2))
    c = jnp.sum(p[1, :, :SEG, :], axis=(0, 2))
    o_ref[...] = s / jnp.maximum(c, 1.0)


def kernel(eps_pred, noise_target, mean, true_mean, variance, true_variance,
           batch):
    N = eps_pred.shape[0]
    CHN = ((N + NW * L - 1) // (NW * L)) * L
    P = NW * CHN - N

    v12 = jnp.stack([eps_pred, noise_target, mean, true_mean, variance,
                     true_variance]).transpose(0, 2, 1).reshape(12, N)
    v12 = jnp.pad(v12, ((0, 0), (0, P)), constant_values=1.0)
    batf = lax.bitcast_convert_type(
        jnp.pad(batch, (0, P), constant_values=SEG), jnp.float32)
    cols = (jnp.concatenate([v12, batf[None]])
            .reshape(NSTR, NW, CHN).transpose(1, 0, 2).reshape(-1))

    partials = _sc_partials_kernel(CHN)(cols)

    return pl.pallas_call(
        _combine_kernel,
        out_shape=jax.ShapeDtypeStruct((SEG,), jnp.float32),
    )(partials)
